# baseline (speedup 1.0000x reference)
import functools

import jax
import jax.numpy as jnp
from jax import lax
from jax.experimental import pallas as pl
from jax.experimental.pallas import tpu as pltpu
from jax.experimental.pallas import tpu_sc as plsc

N = 10000
D_IN = 128
D_MID = 256
D_OUT = 128
K = 27
EPK = 12000
EPS = 1e-5

NC = 2
NS = 16
CHUNKS = 12
CB = 64
NBUF = 4
LEAD = 2
EPK_PAD = NS * CHUNKS * CB
NROW = 10240
ZPT = NROW // NS
K0 = 14

_mesh = plsc.VectorSubcoreMesh(
    core_axis_name="c", subcore_axis_name="s", num_cores=NC, num_subcores=NS)

_sc_scratch = [
    pltpu.VMEM((2, CHUNKS, CB), jnp.int32),
    pltpu.VMEM((2, CHUNKS, CB), jnp.int32),
    pltpu.VMEM((NBUF, CB, D_IN), jnp.float32),
    pltpu.VMEM_SHARED((NROW, D_IN), jnp.float32),
    pltpu.SemaphoreType.DMA,
    pltpu.SemaphoreType.DMA,
    pltpu.SemaphoreType.DMA,
    pltpu.SemaphoreType.DMA,
    pltpu.SemaphoreType.DMA,
    pltpu.SemaphoreType.DMA,
    pltpu.SemaphoreType.DMA,
    pltpu.SemaphoreType.DMA,
    pltpu.SemaphoreType.DMA,
    pltpu.SemaphoreType.DMA,
]


def _fire_idx(srcp_hbm, dstp_hbm, k, s, sidx, didx, slot, isem):
    pltpu.async_copy(srcp_hbm.at[k, s], sidx.at[slot], isem)
    pltpu.async_copy(dstp_hbm.at[k, s], didx.at[slot], isem)


def _wait_idx(srcp_hbm, dstp_hbm, k, s, sidx, didx, slot, isem):
    pltpu.make_async_copy(srcp_hbm.at[k, s], sidx.at[slot], isem).wait()
    pltpu.make_async_copy(dstp_hbm.at[k, s], didx.at[slot], isem).wait()


def _gather_scatter_offset(table, sidx, didx, rows, acc, gsems, ssems,
                           mid=None):
    gd = [pltpu.async_copy(table.at[sidx.at[b]], rows.at[b], gsems[b])
          for b in range(NBUF)]
    if mid is not None:
        mid()
    sd = [None] * NBUF
    for j in range(CHUNKS):
        b = j % NBUF
        f = j + LEAD
        if NBUF <= f < CHUNKS:
            fb = f % NBUF
            sd[fb].wait()
            gd[fb] = pltpu.async_copy(table.at[sidx.at[f]], rows.at[fb],
                                      gsems[fb])
        gd[b].wait()
        sd[b] = pltpu.async_copy(rows.at[b], acc.at[didx.at[j]], ssems[b],
                                 add=True)
    for b in range(NBUF):
        sd[b].wait()


def _make_sc_conv1(base, n0, n1):
    @functools.partial(
        pl.kernel,
        out_type=jax.ShapeDtypeStruct((n0 + n1, NROW, D_IN), jnp.float32),
        mesh=_mesh,
        scratch_types=_sc_scratch,
    )
    def _conv1(x_hbm, srcp_hbm, dstp_hbm, zeros_hbm, y_hbm,
               sidx, didx, rows, acc, isem,
               g0, g1, g2, g3, s0, s1, s2, s3, snap):
        gsems = (g0, g1, g2, g3)
        ssems = (s0, s1, s2, s3)
        c = lax.axis_index("c")
        s = lax.axis_index("s")
        o_lo = jnp.where(c == 0, 0, n0)
        k_lo = base + o_lo
        k_n = jnp.where(c == 0, n0, n1)
        _fire_idx(srcp_hbm, dstp_hbm, k_lo, s, sidx, didx, 0, isem)

        def body(i, carry):
            k = k_lo + i
            o = o_lo + i
            p = i % 2
            _wait_idx(srcp_hbm, dstp_hbm, k, s, sidx, didx, p, isem)

            @pl.when(i + 1 < k_n)
            def _prefetch():
                _fire_idx(srcp_hbm, dstp_hbm, k + 1, s, sidx, didx, 1 - p,
                          isem)

            def _mid():
                @pl.when(i == 0)
                def _():
                    pltpu.sync_copy(zeros_hbm.at[pl.ds(s * ZPT, ZPT)],
                                    acc.at[pl.ds(s * ZPT, ZPT)])

                @pl.when(i > 0)
                def _():
                    pltpu.make_async_copy(
                        acc.at[pl.ds(s * ZPT, ZPT)],
                        y_hbm.at[o, pl.ds(s * ZPT, ZPT)], snap).wait()
                plsc.subcore_barrier()

            _gather_scatter_offset(x_hbm, sidx.at[p], didx.at[p], rows, acc,
                                   gsems, ssems, mid=_mid)
            plsc.subcore_barrier()
            pltpu.async_copy(acc.at[pl.ds(s * ZPT, ZPT)],
                             y_hbm.at[o, pl.ds(s * ZPT, ZPT)], snap)
            return carry

        lax.fori_loop(0, k_n, body, 0)
        pltpu.make_async_copy(acc.at[pl.ds(s * ZPT, ZPT)],
                              y_hbm.at[0, pl.ds(s * ZPT, ZPT)], snap).wait()

    return _conv1


def _make_sc_conv2(base, n0, n1, chained=False):
    @functools.partial(
        pl.kernel,
        out_type=jax.ShapeDtypeStruct((NC, NROW, D_OUT), jnp.float32),
        mesh=_mesh,
        scratch_types=_sc_scratch,
    )
    def _conv2(z_hbm, srcp_hbm, dstp_hbm, init_hbm, out_hbm,
               sidx, didx, rows, acc, isem,
               g0, g1, g2, g3, s0, s1, s2, s3, snap):
        gsems = (g0, g1, g2, g3)
        ssems = (s0, s1, s2, s3)
        c = lax.axis_index("c")
        s = lax.axis_index("s")
        o_lo = jnp.where(c == 0, 0, n0)
        k_lo = base + o_lo
        k_n = jnp.where(c == 0, n0, n1)
        _fire_idx(srcp_hbm, dstp_hbm, k_lo, s, sidx, didx, 0, isem)

        def body(i, carry):
            k = k_lo + i
            o = o_lo + i
            p = i % 2
            _wait_idx(srcp_hbm, dstp_hbm, k, s, sidx, didx, p, isem)

            @pl.when(i + 1 < k_n)
            def _prefetch():
                _fire_idx(srcp_hbm, dstp_hbm, k + 1, s, sidx, didx, 1 - p,
                          isem)

            def _mid():
                @pl.when(i == 0)
                def _():
                    if chained:
                        pltpu.sync_copy(init_hbm.at[c, pl.ds(s * ZPT, ZPT)],
                                        acc.at[pl.ds(s * ZPT, ZPT)])
                    else:
                        pltpu.sync_copy(init_hbm.at[pl.ds(s * ZPT, ZPT)],
                                        acc.at[pl.ds(s * ZPT, ZPT)])
                    plsc.subcore_barrier()

            _gather_scatter_offset(z_hbm.at[o], sidx.at[p], didx.at[p],
                                   rows, acc, gsems, ssems, mid=_mid)
            return carry

        lax.fori_loop(0, k_n, body, 0)
        plsc.subcore_barrier()
        pltpu.sync_copy(acc.at[pl.ds(s * ZPT, ZPT)],
                        out_hbm.at[c, pl.ds(s * ZPT, ZPT)])

    return _conv2


PH_1A = (0, 10, 10)
PH_1B = (20, 4, 3)
N1A = PH_1A[1] + PH_1A[2]
N1B = PH_1B[1] + PH_1B[2]
PH_2A = (0, 3, 3)
PH_2B = (6, 11, 10)
N2A = PH_2A[1] + PH_2A[2]
N2B = PH_2B[1] + PH_2B[2]
_RUN_ENDS = (PH_1A[0] + PH_1A[1] - 1, PH_1A[0] + N1A - 1,
             PH_1B[0] + PH_1B[1] - 1, PH_1B[0] + N1B - 1)

_sc_conv1_a = _make_sc_conv1(*PH_1A)
_sc_conv1_b = _make_sc_conv1(*PH_1B)
_sc_conv2_a = _make_sc_conv2(*PH_2A)
_sc_conv2_b = _make_sc_conv2(*PH_2B, chained=True)


def _tc_gemm1a_body(y_ref, w_ref, acc_ref):
    k = pl.program_id(0)

    @pl.when(k == 0)
    def _init():
        acc_ref[...] = jnp.zeros_like(acc_ref)

    acc_ref[...] += jnp.dot(y_ref[0], w_ref[0],
                            preferred_element_type=jnp.float32)


def _tc_gemm1b_body(y_ref, w_ref, h1a_ref, b_ref, g_ref, be_ref, h_ref,
                    acc_ref):
    k = pl.program_id(0)

    @pl.when(k == 0)
    def _init():
        acc_ref[...] = h1a_ref[...]

    acc_ref[...] += jnp.dot(y_ref[0], w_ref[0],
                            preferred_element_type=jnp.float32)

    @pl.when(k == N1B - 1)
    def _fin():
        h = acc_ref[...] + b_ref[...]
        m = jnp.mean(h, axis=0, keepdims=True)
        hc = h - m
        v = jnp.mean(hc * hc, axis=0, keepdims=True)
        h = hc * lax.rsqrt(v + EPS) * g_ref[...] + be_ref[...]
        h_ref[...] = jnp.maximum(h, 0.0)


def _tc_gemm2_body(h_ref, w_ref, z_ref):
    z_ref[0] = jnp.dot(h_ref[...], w_ref[0],
                       preferred_element_type=jnp.float32)


def _tc_final_body(pb_ref, x_ref, b_ref, g_ref, be_ref, o_ref):
    h = pb_ref[0] + pb_ref[1] + b_ref[...]
    m = jnp.mean(h, axis=0, keepdims=True)
    hc = h - m
    v = jnp.mean(hc * hc, axis=0, keepdims=True)
    h = hc * lax.rsqrt(v + EPS) * g_ref[...] + be_ref[...] + x_ref[...]
    o_ref[...] = jnp.maximum(h, 0.0)


def _gemm1_call(body, nk, extra_specs, out_shape, scratch=()):
    return pl.pallas_call(
        body,
        grid=(nk,),
        in_specs=[
            pl.BlockSpec((1, N, D_IN), lambda k: (k, 0, 0)),
            pl.BlockSpec((1, D_IN, D_MID), lambda k: (k, 0, 0)),
        ] + extra_specs,
        out_specs=pl.BlockSpec((N, D_MID), lambda k: (0, 0)),
        out_shape=out_shape,
        scratch_shapes=list(scratch),
    )


def _gemm2_call(nk):
    return pl.pallas_call(
        _tc_gemm2_body,
        grid=(nk,),
        in_specs=[
            pl.BlockSpec((N, D_MID), lambda k: (0, 0)),
            pl.BlockSpec((1, D_MID, D_OUT), lambda k: (k, 0, 0)),
        ],
        out_specs=pl.BlockSpec((1, N, D_OUT), lambda k: (k, 0, 0)),
        out_shape=jax.ShapeDtypeStruct((nk, N, D_OUT), jnp.float32),
    )


def kernel(x, edge_index, W1, bc1, g1, be1, W2, bc2, g2, be2):
    src = edge_index[0].astype(jnp.int32).reshape(K, EPK)
    dst = edge_index[1].astype(jnp.int32).reshape(K, EPK)
    pad = EPK_PAD - EPK
    srcp = jnp.pad(src, ((0, 0), (0, pad))).reshape(K, NS, CHUNKS, CB)
    dstp = jnp.pad(dst, ((0, 0), (0, pad)),
                   constant_values=N).reshape(K, NS, CHUNKS, CB)
    zeros = jnp.zeros((NROW, D_IN), jnp.float32)

    W1n = jnp.concatenate([W1[1:], jnp.zeros_like(W1[:1])], axis=0)
    run_mask = jnp.asarray(
        [0.0 if k in _RUN_ENDS else 1.0 for k in range(K)],
        jnp.float32).reshape(K, 1, 1)
    W1eff = W1 - run_mask * W1n

    ya = _sc_conv1_a(x, srcp, dstp, zeros)
    yb = _sc_conv1_b(x, srcp, dstp, zeros)
    h1a = _gemm1_call(
        _tc_gemm1a_body, N1A, [],
        jax.ShapeDtypeStruct((N, D_MID), jnp.float32),
    )(ya, W1eff[:N1A])
    h = _gemm1_call(
        _tc_gemm1b_body, N1B,
        [
            pl.BlockSpec((N, D_MID), lambda k: (0, 0)),
            pl.BlockSpec((1, D_MID), lambda k: (0, 0)),
            pl.BlockSpec((1, D_MID), lambda k: (0, 0)),
            pl.BlockSpec((1, D_MID), lambda k: (0, 0)),
        ],
        jax.ShapeDtypeStruct((N, D_MID), jnp.float32),
        scratch=[pltpu.VMEM((N, D_MID), jnp.float32)],
    )(yb, W1eff[N1A:], h1a, bc1.reshape(1, -1), g1.reshape(1, -1),
      be1.reshape(1, -1))

    za = _gemm2_call(N2A)(h, W2[:N2A])
    zb = _gemm2_call(N2B)(h, W2[N2A:])
    pa = _sc_conv2_a(za, srcp, dstp, zeros)
    pb = _sc_conv2_b(zb, srcp, dstp, pa)

    out = pl.pallas_call(
        _tc_final_body,
        grid=(1,),
        in_specs=[
            pl.BlockSpec((NC, N, D_OUT), lambda i: (0, 0, 0)),
            pl.BlockSpec((N, D_OUT), lambda i: (0, 0)),
            pl.BlockSpec((1, D_OUT), lambda i: (0, 0)),
            pl.BlockSpec((1, D_OUT), lambda i: (0, 0)),
            pl.BlockSpec((1, D_OUT), lambda i: (0, 0)),
        ],
        out_specs=pl.BlockSpec((N, D_OUT), lambda i: (0, 0)),
        out_shape=jax.ShapeDtypeStruct((N, D_OUT), jnp.float32),
    )(pb, x, bc2.reshape(1, -1), g2.reshape(1, -1), be2.reshape(1, -1))

    return out


# speedup vs baseline: 1.0217x; 1.0217x over previous
import functools

import jax
import jax.numpy as jnp
from jax import lax
from jax.experimental import pallas as pl
from jax.experimental.pallas import tpu as pltpu
from jax.experimental.pallas import tpu_sc as plsc

N = 10000
D_IN = 128
D_MID = 256
D_OUT = 128
K = 27
EPK = 12000
EPS = 1e-5

NC = 2
NS = 16
CHUNKS = 12
CB = 64
NBUF = 4
LEAD = 2
EPK_PAD = NS * CHUNKS * CB
NROW = 10240
ZPT = NROW // NS
K0 = 14

_mesh = plsc.VectorSubcoreMesh(
    core_axis_name="c", subcore_axis_name="s", num_cores=NC, num_subcores=NS)

_sc_scratch = [
    pltpu.VMEM((2, CHUNKS, CB), jnp.int32),
    pltpu.VMEM((2, CHUNKS, CB), jnp.int32),
    pltpu.VMEM((NBUF, CB, D_IN), jnp.float32),
    pltpu.VMEM_SHARED((NROW, D_IN), jnp.float32),
    pltpu.SemaphoreType.DMA,
    pltpu.SemaphoreType.DMA,
    pltpu.SemaphoreType.DMA,
    pltpu.SemaphoreType.DMA,
    pltpu.SemaphoreType.DMA,
    pltpu.SemaphoreType.DMA,
    pltpu.SemaphoreType.DMA,
    pltpu.SemaphoreType.DMA,
    pltpu.SemaphoreType.DMA,
    pltpu.SemaphoreType.DMA,
]


def _fire_idx(srcp_hbm, dstp_hbm, k, s, sidx, didx, slot, isem):
    pltpu.async_copy(srcp_hbm.at[k, s], sidx.at[slot], isem)
    pltpu.async_copy(dstp_hbm.at[k, s], didx.at[slot], isem)


def _wait_idx(srcp_hbm, dstp_hbm, k, s, sidx, didx, slot, isem):
    pltpu.make_async_copy(srcp_hbm.at[k, s], sidx.at[slot], isem).wait()
    pltpu.make_async_copy(dstp_hbm.at[k, s], didx.at[slot], isem).wait()


def _gather_scatter_offset(table, sidx, didx, rows, acc, gsems, ssems,
                           mid=None):
    gd = [pltpu.async_copy(table.at[sidx.at[b]], rows.at[b], gsems[b])
          for b in range(NBUF)]
    if mid is not None:
        mid()
    sd = [None] * NBUF
    for j in range(CHUNKS):
        b = j % NBUF
        f = j + LEAD
        if NBUF <= f < CHUNKS:
            fb = f % NBUF
            sd[fb].wait()
            gd[fb] = pltpu.async_copy(table.at[sidx.at[f]], rows.at[fb],
                                      gsems[fb])
        gd[b].wait()
        sd[b] = pltpu.async_copy(rows.at[b], acc.at[didx.at[j]], ssems[b],
                                 add=True)
    for b in range(NBUF):
        sd[b].wait()


def _make_sc_conv1(base, n0, n1):
    @functools.partial(
        pl.kernel,
        out_type=jax.ShapeDtypeStruct((n0 + n1, NROW, D_IN), jnp.float32),
        mesh=_mesh,
        scratch_types=_sc_scratch,
    )
    def _conv1(x_hbm, srcp_hbm, dstp_hbm, zeros_hbm, y_hbm,
               sidx, didx, rows, acc, isem,
               g0, g1, g2, g3, s0, s1, s2, s3, snap):
        gsems = (g0, g1, g2, g3)
        ssems = (s0, s1, s2, s3)
        c = lax.axis_index("c")
        s = lax.axis_index("s")
        o_lo = jnp.where(c == 0, 0, n0)
        k_lo = base + o_lo
        k_n = jnp.where(c == 0, n0, n1)
        _fire_idx(srcp_hbm, dstp_hbm, k_lo, s, sidx, didx, 0, isem)

        def body(i, carry):
            k = k_lo + i
            o = o_lo + i
            p = i % 2
            _wait_idx(srcp_hbm, dstp_hbm, k, s, sidx, didx, p, isem)

            @pl.when(i + 1 < k_n)
            def _prefetch():
                _fire_idx(srcp_hbm, dstp_hbm, k + 1, s, sidx, didx, 1 - p,
                          isem)

            def _mid():
                @pl.when(i == 0)
                def _():
                    pltpu.sync_copy(zeros_hbm.at[pl.ds(s * ZPT, ZPT)],
                                    acc.at[pl.ds(s * ZPT, ZPT)])

                @pl.when(i > 0)
                def _():
                    pltpu.make_async_copy(
                        acc.at[pl.ds(s * ZPT, ZPT)],
                        y_hbm.at[o, pl.ds(s * ZPT, ZPT)], snap).wait()
                plsc.subcore_barrier()

            _gather_scatter_offset(x_hbm, sidx.at[p], didx.at[p], rows, acc,
                                   gsems, ssems, mid=_mid)
            plsc.subcore_barrier()
            pltpu.async_copy(acc.at[pl.ds(s * ZPT, ZPT)],
                             y_hbm.at[o, pl.ds(s * ZPT, ZPT)], snap)
            return carry

        lax.fori_loop(0, k_n, body, 0)
        pltpu.make_async_copy(acc.at[pl.ds(s * ZPT, ZPT)],
                              y_hbm.at[0, pl.ds(s * ZPT, ZPT)], snap).wait()

    return _conv1


def _make_sc_conv2(base, n0, n1, chained=False):
    @functools.partial(
        pl.kernel,
        out_type=jax.ShapeDtypeStruct((NC, NROW, D_OUT), jnp.float32),
        mesh=_mesh,
        scratch_types=_sc_scratch,
    )
    def _conv2(z_hbm, srcp_hbm, dstp_hbm, init_hbm, out_hbm,
               sidx, didx, rows, acc, isem,
               g0, g1, g2, g3, s0, s1, s2, s3, snap):
        gsems = (g0, g1, g2, g3)
        ssems = (s0, s1, s2, s3)
        c = lax.axis_index("c")
        s = lax.axis_index("s")
        o_lo = jnp.where(c == 0, 0, n0)
        k_lo = base + o_lo
        k_n = jnp.where(c == 0, n0, n1)
        _fire_idx(srcp_hbm, dstp_hbm, k_lo, s, sidx, didx, 0, isem)

        def body(i, carry):
            k = k_lo + i
            o = o_lo + i
            p = i % 2
            _wait_idx(srcp_hbm, dstp_hbm, k, s, sidx, didx, p, isem)

            @pl.when(i + 1 < k_n)
            def _prefetch():
                _fire_idx(srcp_hbm, dstp_hbm, k + 1, s, sidx, didx, 1 - p,
                          isem)

            def _mid():
                @pl.when(i == 0)
                def _():
                    if chained:
                        pltpu.sync_copy(init_hbm.at[c, pl.ds(s * ZPT, ZPT)],
                                        acc.at[pl.ds(s * ZPT, ZPT)])
                    else:
                        pltpu.sync_copy(init_hbm.at[pl.ds(s * ZPT, ZPT)],
                                        acc.at[pl.ds(s * ZPT, ZPT)])
                    plsc.subcore_barrier()

            _gather_scatter_offset(z_hbm.at[o], sidx.at[p], didx.at[p],
                                   rows, acc, gsems, ssems, mid=_mid)
            return carry

        lax.fori_loop(0, k_n, body, 0)
        plsc.subcore_barrier()
        pltpu.sync_copy(acc.at[pl.ds(s * ZPT, ZPT)],
                        out_hbm.at[c, pl.ds(s * ZPT, ZPT)])

    return _conv2


K0 = 14
_RUN_ENDS = (K0 - 1, K - 1)

_sc_conv1 = _make_sc_conv1(0, K0, K - K0)
_sc_conv2 = _make_sc_conv2(0, K0, K - K0)


def _tc_gemm1_body(y_ref, w_ref, b_ref, g_ref, be_ref, h_ref, acc_ref):
    k = pl.program_id(0)

    @pl.when(k == 0)
    def _init():
        acc_ref[...] = jnp.zeros_like(acc_ref)

    acc_ref[...] += jnp.dot(y_ref[0], w_ref[0],
                            preferred_element_type=jnp.float32)

    @pl.when(k == K - 1)
    def _fin():
        h = acc_ref[...] + b_ref[...]
        m = jnp.mean(h, axis=0, keepdims=True)
        hc = h - m
        v = jnp.mean(hc * hc, axis=0, keepdims=True)
        h = hc * lax.rsqrt(v + EPS) * g_ref[...] + be_ref[...]
        h_ref[...] = jnp.maximum(h, 0.0)


def _tc_gemm2_body(h_ref, w_ref, z_ref):
    z_ref[0] = jnp.dot(h_ref[...], w_ref[0],
                       preferred_element_type=jnp.float32)


def _tc_final_body(pb_ref, x_ref, b_ref, g_ref, be_ref, o_ref):
    h = pb_ref[0] + pb_ref[1] + b_ref[...]
    m = jnp.mean(h, axis=0, keepdims=True)
    hc = h - m
    v = jnp.mean(hc * hc, axis=0, keepdims=True)
    h = hc * lax.rsqrt(v + EPS) * g_ref[...] + be_ref[...] + x_ref[...]
    o_ref[...] = jnp.maximum(h, 0.0)


def _gemm1_call(body, nk, extra_specs, out_shape, scratch=()):
    return pl.pallas_call(
        body,
        grid=(nk,),
        in_specs=[
            pl.BlockSpec((1, N, D_IN), lambda k: (k, 0, 0)),
            pl.BlockSpec((1, D_IN, D_MID), lambda k: (k, 0, 0)),
        ] + extra_specs,
        out_specs=pl.BlockSpec((N, D_MID), lambda k: (0, 0)),
        out_shape=out_shape,
        scratch_shapes=list(scratch),
    )


def _gemm2_call(nk):
    return pl.pallas_call(
        _tc_gemm2_body,
        grid=(nk,),
        in_specs=[
            pl.BlockSpec((N, D_MID), lambda k: (0, 0)),
            pl.BlockSpec((1, D_MID, D_OUT), lambda k: (k, 0, 0)),
        ],
        out_specs=pl.BlockSpec((1, N, D_OUT), lambda k: (k, 0, 0)),
        out_shape=jax.ShapeDtypeStruct((nk, N, D_OUT), jnp.float32),
    )


def kernel(x, edge_index, W1, bc1, g1, be1, W2, bc2, g2, be2):
    src = edge_index[0].astype(jnp.int32).reshape(K, EPK)
    dst = edge_index[1].astype(jnp.int32).reshape(K, EPK)
    pad = EPK_PAD - EPK
    srcp = jnp.pad(src, ((0, 0), (0, pad))).reshape(K, NS, CHUNKS, CB)
    dstp = jnp.pad(dst, ((0, 0), (0, pad)),
                   constant_values=N).reshape(K, NS, CHUNKS, CB)
    zeros = jnp.zeros((NROW, D_IN), jnp.float32)

    W1n = jnp.concatenate([W1[1:], jnp.zeros_like(W1[:1])], axis=0)
    run_mask = jnp.asarray(
        [0.0 if k in _RUN_ENDS else 1.0 for k in range(K)],
        jnp.float32).reshape(K, 1, 1)
    W1eff = W1 - run_mask * W1n

    y = _sc_conv1(x, srcp, dstp, zeros)
    h = _gemm1_call(
        _tc_gemm1_body, K,
        [
            pl.BlockSpec((1, D_MID), lambda k: (0, 0)),
            pl.BlockSpec((1, D_MID), lambda k: (0, 0)),
            pl.BlockSpec((1, D_MID), lambda k: (0, 0)),
        ],
        jax.ShapeDtypeStruct((N, D_MID), jnp.float32),
        scratch=[pltpu.VMEM((N, D_MID), jnp.float32)],
    )(y, W1eff, bc1.reshape(1, -1), g1.reshape(1, -1), be1.reshape(1, -1))

    z = _gemm2_call(K)(h, W2)
    pb = _sc_conv2(z, srcp, dstp, zeros)

    out = pl.pallas_call(
        _tc_final_body,
        grid=(1,),
        in_specs=[
            pl.BlockSpec((NC, N, D_OUT), lambda i: (0, 0, 0)),
            pl.BlockSpec((N, D_OUT), lambda i: (0, 0)),
            pl.BlockSpec((1, D_OUT), lambda i: (0, 0)),
            pl.BlockSpec((1, D_OUT), lambda i: (0, 0)),
            pl.BlockSpec((1, D_OUT), lambda i: (0, 0)),
        ],
        out_specs=pl.BlockSpec((N, D_OUT), lambda i: (0, 0)),
        out_shape=jax.ShapeDtypeStruct((N, D_OUT), jnp.float32),
    )(pb, x, bc2.reshape(1, -1), g2.reshape(1, -1), be2.reshape(1, -1))

    return out


# speedup vs baseline: 1.0265x; 1.0047x over previous
import functools

import jax
import jax.numpy as jnp
from jax import lax
from jax.experimental import pallas as pl
from jax.experimental.pallas import tpu as pltpu
from jax.experimental.pallas import tpu_sc as plsc

N = 10000
D_IN = 128
D_MID = 256
D_OUT = 128
K = 27
EPK = 12000
EPS = 1e-5

NC = 2
NS = 16
CHUNKS = 6
CB = 128
NBUF = 2
LEAD = 1
EPK_PAD = NS * CHUNKS * CB
NROW = 10240
ZPT = NROW // NS
K0 = 14

_mesh = plsc.VectorSubcoreMesh(
    core_axis_name="c", subcore_axis_name="s", num_cores=NC, num_subcores=NS)

_sc_scratch = [
    pltpu.VMEM((2, CHUNKS, CB), jnp.int32),
    pltpu.VMEM((2, CHUNKS, CB), jnp.int32),
    pltpu.VMEM((NBUF, CB, D_IN), jnp.float32),
    pltpu.VMEM_SHARED((NROW, D_IN), jnp.float32),
    pltpu.SemaphoreType.DMA,
    pltpu.SemaphoreType.DMA,
    pltpu.SemaphoreType.DMA,
    pltpu.SemaphoreType.DMA,
    pltpu.SemaphoreType.DMA,
    pltpu.SemaphoreType.DMA,
    pltpu.SemaphoreType.DMA,
    pltpu.SemaphoreType.DMA,
    pltpu.SemaphoreType.DMA,
    pltpu.SemaphoreType.DMA,
]


def _fire_idx(srcp_hbm, dstp_hbm, k, s, sidx, didx, slot, isem):
    pltpu.async_copy(srcp_hbm.at[k, s], sidx.at[slot], isem)
    pltpu.async_copy(dstp_hbm.at[k, s], didx.at[slot], isem)


def _wait_idx(srcp_hbm, dstp_hbm, k, s, sidx, didx, slot, isem):
    pltpu.make_async_copy(srcp_hbm.at[k, s], sidx.at[slot], isem).wait()
    pltpu.make_async_copy(dstp_hbm.at[k, s], didx.at[slot], isem).wait()


def _gather_scatter_offset(table, sidx, didx, rows, acc, gsems, ssems,
                           mid=None):
    gd = [pltpu.async_copy(table.at[sidx.at[b]], rows.at[b], gsems[b])
          for b in range(NBUF)]
    if mid is not None:
        mid()
    sd = [None] * NBUF
    for j in range(CHUNKS):
        b = j % NBUF
        f = j + LEAD
        if NBUF <= f < CHUNKS:
            fb = f % NBUF
            sd[fb].wait()
            gd[fb] = pltpu.async_copy(table.at[sidx.at[f]], rows.at[fb],
                                      gsems[fb])
        gd[b].wait()
        sd[b] = pltpu.async_copy(rows.at[b], acc.at[didx.at[j]], ssems[b],
                                 add=True)
    for b in range(NBUF):
        sd[b].wait()


def _make_sc_conv1(base, n0, n1):
    @functools.partial(
        pl.kernel,
        out_type=jax.ShapeDtypeStruct((n0 + n1, NROW, D_IN), jnp.float32),
        mesh=_mesh,
        scratch_types=_sc_scratch,
    )
    def _conv1(x_hbm, srcp_hbm, dstp_hbm, zeros_hbm, y_hbm,
               sidx, didx, rows, acc, isem,
               g0, g1, g2, g3, s0, s1, s2, s3, snap):
        gsems = (g0, g1, g2, g3)
        ssems = (s0, s1, s2, s3)
        c = lax.axis_index("c")
        s = lax.axis_index("s")
        o_lo = jnp.where(c == 0, 0, n0)
        k_lo = base + o_lo
        k_n = jnp.where(c == 0, n0, n1)
        _fire_idx(srcp_hbm, dstp_hbm, k_lo, s, sidx, didx, 0, isem)

        def body(i, carry):
            k = k_lo + i
            o = o_lo + i
            p = i % 2
            _wait_idx(srcp_hbm, dstp_hbm, k, s, sidx, didx, p, isem)

            @pl.when(i + 1 < k_n)
            def _prefetch():
                _fire_idx(srcp_hbm, dstp_hbm, k + 1, s, sidx, didx, 1 - p,
                          isem)

            def _mid():
                @pl.when(i == 0)
                def _():
                    pltpu.sync_copy(zeros_hbm.at[pl.ds(s * ZPT, ZPT)],
                                    acc.at[pl.ds(s * ZPT, ZPT)])

                @pl.when(i > 0)
                def _():
                    pltpu.make_async_copy(
                        acc.at[pl.ds(s * ZPT, ZPT)],
                        y_hbm.at[o, pl.ds(s * ZPT, ZPT)], snap).wait()
                plsc.subcore_barrier()

            _gather_scatter_offset(x_hbm, sidx.at[p], didx.at[p], rows, acc,
                                   gsems, ssems, mid=_mid)
            plsc.subcore_barrier()
            pltpu.async_copy(acc.at[pl.ds(s * ZPT, ZPT)],
                             y_hbm.at[o, pl.ds(s * ZPT, ZPT)], snap)
            return carry

        lax.fori_loop(0, k_n, body, 0)
        pltpu.make_async_copy(acc.at[pl.ds(s * ZPT, ZPT)],
                              y_hbm.at[0, pl.ds(s * ZPT, ZPT)], snap).wait()

    return _conv1


def _make_sc_conv2(base, n0, n1, chained=False):
    @functools.partial(
        pl.kernel,
        out_type=jax.ShapeDtypeStruct((NC, NROW, D_OUT), jnp.float32),
        mesh=_mesh,
        scratch_types=_sc_scratch,
    )
    def _conv2(z_hbm, srcp_hbm, dstp_hbm, init_hbm, out_hbm,
               sidx, didx, rows, acc, isem,
               g0, g1, g2, g3, s0, s1, s2, s3, snap):
        gsems = (g0, g1, g2, g3)
        ssems = (s0, s1, s2, s3)
        c = lax.axis_index("c")
        s = lax.axis_index("s")
        o_lo = jnp.where(c == 0, 0, n0)
        k_lo = base + o_lo
        k_n = jnp.where(c == 0, n0, n1)
        _fire_idx(srcp_hbm, dstp_hbm, k_lo, s, sidx, didx, 0, isem)

        def body(i, carry):
            k = k_lo + i
            o = o_lo + i
            p = i % 2
            _wait_idx(srcp_hbm, dstp_hbm, k, s, sidx, didx, p, isem)

            @pl.when(i + 1 < k_n)
            def _prefetch():
                _fire_idx(srcp_hbm, dstp_hbm, k + 1, s, sidx, didx, 1 - p,
                          isem)

            def _mid():
                @pl.when(i == 0)
                def _():
                    if chained:
                        pltpu.sync_copy(init_hbm.at[c, pl.ds(s * ZPT, ZPT)],
                                        acc.at[pl.ds(s * ZPT, ZPT)])
                    else:
                        pltpu.sync_copy(init_hbm.at[pl.ds(s * ZPT, ZPT)],
                                        acc.at[pl.ds(s * ZPT, ZPT)])
                    plsc.subcore_barrier()

            _gather_scatter_offset(z_hbm.at[o], sidx.at[p], didx.at[p],
                                   rows, acc, gsems, ssems, mid=_mid)
            return carry

        lax.fori_loop(0, k_n, body, 0)
        plsc.subcore_barrier()
        pltpu.sync_copy(acc.at[pl.ds(s * ZPT, ZPT)],
                        out_hbm.at[c, pl.ds(s * ZPT, ZPT)])

    return _conv2


K0 = 14
_RUN_ENDS = (K0 - 1, K - 1)

_sc_conv1 = _make_sc_conv1(0, K0, K - K0)
_sc_conv2 = _make_sc_conv2(0, K0, K - K0)


def _tc_gemm1_body(y_ref, w_ref, b_ref, g_ref, be_ref, h_ref, acc_ref):
    k = pl.program_id(0)

    @pl.when(k == 0)
    def _init():
        acc_ref[...] = jnp.zeros_like(acc_ref)

    acc_ref[...] += jnp.dot(y_ref[0], w_ref[0],
                            preferred_element_type=jnp.float32)

    @pl.when(k == K - 1)
    def _fin():
        h = acc_ref[...] + b_ref[...]
        m = jnp.mean(h, axis=0, keepdims=True)
        hc = h - m
        v = jnp.mean(hc * hc, axis=0, keepdims=True)
        h = hc * lax.rsqrt(v + EPS) * g_ref[...] + be_ref[...]
        h_ref[...] = jnp.maximum(h, 0.0)


def _tc_gemm2_body(h_ref, w_ref, z_ref):
    z_ref[0] = jnp.dot(h_ref[...], w_ref[0],
                       preferred_element_type=jnp.float32)


def _tc_final_body(pb_ref, x_ref, b_ref, g_ref, be_ref, o_ref):
    h = pb_ref[0] + pb_ref[1] + b_ref[...]
    m = jnp.mean(h, axis=0, keepdims=True)
    hc = h - m
    v = jnp.mean(hc * hc, axis=0, keepdims=True)
    h = hc * lax.rsqrt(v + EPS) * g_ref[...] + be_ref[...] + x_ref[...]
    o_ref[...] = jnp.maximum(h, 0.0)


def _gemm1_call(body, nk, extra_specs, out_shape, scratch=()):
    return pl.pallas_call(
        body,
        grid=(nk,),
        in_specs=[
            pl.BlockSpec((1, N, D_IN), lambda k: (k, 0, 0)),
            pl.BlockSpec((1, D_IN, D_MID), lambda k: (k, 0, 0)),
        ] + extra_specs,
        out_specs=pl.BlockSpec((N, D_MID), lambda k: (0, 0)),
        out_shape=out_shape,
        scratch_shapes=list(scratch),
    )


def _gemm2_call(nk):
    return pl.pallas_call(
        _tc_gemm2_body,
        grid=(nk,),
        in_specs=[
            pl.BlockSpec((N, D_MID), lambda k: (0, 0)),
            pl.BlockSpec((1, D_MID, D_OUT), lambda k: (k, 0, 0)),
        ],
        out_specs=pl.BlockSpec((1, N, D_OUT), lambda k: (k, 0, 0)),
        out_shape=jax.ShapeDtypeStruct((nk, N, D_OUT), jnp.float32),
    )


def kernel(x, edge_index, W1, bc1, g1, be1, W2, bc2, g2, be2):
    src = edge_index[0].astype(jnp.int32).reshape(K, EPK)
    dst = edge_index[1].astype(jnp.int32).reshape(K, EPK)
    pad = EPK_PAD - EPK
    srcp = jnp.pad(src, ((0, 0), (0, pad))).reshape(K, NS, CHUNKS, CB)
    dstp = jnp.pad(dst, ((0, 0), (0, pad)),
                   constant_values=N).reshape(K, NS, CHUNKS, CB)
    zeros = jnp.zeros((NROW, D_IN), jnp.float32)

    W1n = jnp.concatenate([W1[1:], jnp.zeros_like(W1[:1])], axis=0)
    run_mask = jnp.asarray(
        [0.0 if k in _RUN_ENDS else 1.0 for k in range(K)],
        jnp.float32).reshape(K, 1, 1)
    W1eff = W1 - run_mask * W1n

    y = _sc_conv1(x, srcp, dstp, zeros)
    h = _gemm1_call(
        _tc_gemm1_body, K,
        [
            pl.BlockSpec((1, D_MID), lambda k: (0, 0)),
            pl.BlockSpec((1, D_MID), lambda k: (0, 0)),
            pl.BlockSpec((1, D_MID), lambda k: (0, 0)),
        ],
        jax.ShapeDtypeStruct((N, D_MID), jnp.float32),
        scratch=[pltpu.VMEM((N, D_MID), jnp.float32)],
    )(y, W1eff, bc1.reshape(1, -1), g1.reshape(1, -1), be1.reshape(1, -1))

    z = _gemm2_call(K)(h, W2)
    pb = _sc_conv2(z, srcp, dstp, zeros)

    out = pl.pallas_call(
        _tc_final_body,
        grid=(1,),
        in_specs=[
            pl.BlockSpec((NC, N, D_OUT), lambda i: (0, 0, 0)),
            pl.BlockSpec((N, D_OUT), lambda i: (0, 0)),
            pl.BlockSpec((1, D_OUT), lambda i: (0, 0)),
            pl.BlockSpec((1, D_OUT), lambda i: (0, 0)),
            pl.BlockSpec((1, D_OUT), lambda i: (0, 0)),
        ],
        out_specs=pl.BlockSpec((N, D_OUT), lambda i: (0, 0)),
        out_shape=jax.ShapeDtypeStruct((N, D_OUT), jnp.float32),
    )(pb, x, bc2.reshape(1, -1), g2.reshape(1, -1), be2.reshape(1, -1))

    return out
